# Initial kernel scaffold; baseline (speedup 1.0000x reference)
import jax
import jax.numpy as jnp
from jax.experimental import pallas as pl


def kernel(obs, edge_index, s1_tc1_W, s1_tc1_b, s1_cheb_W, s1_cheb_b, s1_tc2_W, s1_tc2_b, s1_bn_g, s1_bn_b, s2_tc1_W, s2_tc1_b, s2_cheb_W, s2_cheb_b, s2_tc2_W, s2_tc2_b, s2_bn_g, s2_bn_b, lin_W, lin_b):
    raise NotImplementedError("write your pallas kernel here")


# speedup vs baseline: 1.9926x; 1.9926x over previous
import functools

import jax
import jax.numpy as jnp
from jax import lax
from jax.experimental import pallas as pl
from jax.experimental.pallas import tpu as pltpu
from jax.experimental.pallas import tpu_sc as plsc

_N = 11
_E = 110
_H = 32
_EPS = 1e-5
_L = 16

_f32 = jnp.float32
_i32 = jnp.int32

_GDN = lax.GatherDimensionNumbers(
    offset_dims=(), collapsed_slice_dims=(0,), start_index_map=(0,))


def _bc(v, j):
    idx = jnp.full((_L, 1), j, _i32)
    return lax.gather(v, idx, dimension_numbers=_GDN, slice_sizes=(1,),
                      mode=lax.GatherScatterMode.PROMISE_IN_BOUNDS)


def _full(x):
    return jnp.full((_L,), x, _i32)


def _rsqrt(x):
    i = lax.bitcast_convert_type(x, _i32)
    i = jnp.int32(0x5F3759DF) - lax.shift_right_logical(i, 1)
    y = lax.bitcast_convert_type(i, _f32)
    for _ in range(3):
        y = y * (1.5 - 0.5 * x * y * y)
    return y


def _sig(z):
    return 1.0 / (1.0 + jnp.exp(-z))


def _body(obs_h, ei_h,
          w1a_h, b1a_h, wc1_h, bc1_h, w1b_h, b1b_h, g1_h, be1_h,
          w2a_h, b2a_h, wc2_h, bc2_h, w2b_h, b2b_h, g2_h, be2_h,
          lw_h, lb_h, out_h,
          obs_v, ei_v,
          w1a_v, b1a_v, wc1_v, bc1_v, w1b_v, b1b_v, g1_v, be1_v,
          w2a_v, b2a_v, wc2_v, bc2_v, w2b_v, b2b_v, g2_v, be2_v,
          lw_v, lb_v,
          t0_v, tx1_v, tx2_v, tc_v, cnt_v, ah_v, idx_v, dis_v, ob_v,
          sem):
    c_id = lax.axis_index("c")
    s_id = lax.axis_index("s")

    @pl.when(jnp.logical_and(c_id == 0, s_id == 0))
    def _run():
        iota = lax.iota(_i32, _L)
        ci = jnp.minimum(iota, _N - 1)
        i12 = jnp.minimum(iota, 11) * 12
        i32c = ci * _H
        zero = jnp.zeros((_L,), _f32)
        ones = jnp.ones((_L,), _f32)
        lane0 = iota == 0

        cp_ei = pltpu.async_copy(ei_h, ei_v, sem)
        pairs = ((obs_h, obs_v),
                 (w1a_h, w1a_v), (b1a_h, b1a_v), (wc1_h, wc1_v),
                 (bc1_h, bc1_v), (w1b_h, w1b_v), (b1b_h, b1b_v),
                 (g1_h, g1_v), (be1_h, be1_v),
                 (w2a_h, w2a_v), (b2a_h, b2a_v), (wc2_h, wc2_v),
                 (bc2_h, bc2_v), (w2b_h, w2b_v), (b2b_h, b2b_v),
                 (g2_h, g2_v), (be2_h, be2_v),
                 (lw_h, lw_v), (lb_h, lb_v))
        cps = [pltpu.async_copy(src, dst, sem) for src, dst in pairs]
        cp_ei.wait()

        for c in range(7):
            eidx = jnp.minimum(iota + 16 * c, _E - 1)
            r = plsc.load_gather(ei_v, [eidx])
            co = plsc.load_gather(ei_v, [eidx + _E])
            idx_v[pl.ds(16 * c, 16)] = co * 12 + r

        for v in range(9):
            cnt_v[pl.ds(16 * v, 16)] = zero

        def _cbody(e, carry):
            i = plsc.load_gather(idx_v, [_full(0) + e])
            plsc.addupdate_scatter(cnt_v, [i], ones, mask=lane0)
            return carry

        lax.fori_loop(0, _E, _cbody, 0)

        deg = zero
        for d in range(_N):
            deg = deg + plsc.load_gather(cnt_v, [_full(d * 12) + iota])
        dis_v[pl.ds(0, 16)] = jnp.where(deg > 0.0, _rsqrt(deg), 0.0)

        for v in range(9):
            l = iota + 16 * v
            dd = lax.div(l, _full(12))
            ss = lax.rem(l, _full(12))
            a = -(plsc.load_gather(dis_v, [dd]) * plsc.load_gather(dis_v, [ss]))
            ah_v[pl.ds(16 * v, 16)] = a * cnt_v[pl.ds(16 * v, 16)]

        for cp in cps:
            cp.wait()

        def store22(ref, vals):
            for d in range(_N):
                ref[pl.ds(d * _H, 16)] = vals[2 * d]
                ref[pl.ds(d * _H + 16, 16)] = vals[2 * d + 1]

        def prop(src):
            def pbody(s, acc):
                acol = plsc.load_gather(ah_v, [i12 + s])
                vlo = src[pl.ds(s * _H, 16)]
                vhi = src[pl.ds(s * _H + 16, 16)]
                out = []
                for d in range(_N):
                    ab = _bc(acol, d)
                    out.append(acc[2 * d] + ab * vlo)
                    out.append(acc[2 * d + 1] + ab * vhi)
                return tuple(out)

            return lax.fori_loop(0, _N, pbody, (zero,) * 22)

        def tconv1(x0, x1, w_v, b_v, dst):
            w = [[[plsc.load_gather(w_v, [(iota + 16 * h) * 2 + (j * 64 + c)])
                   for h in (0, 1)] for c in (0, 1)] for j in range(3)]
            b = [[b_v[pl.ds(j * _H + 16 * h, 16)] for h in (0, 1)]
                 for j in range(3)]
            vals = []
            for n in range(_N):
                xb0 = _bc(x0, n)
                xb1 = _bc(x1, n)
                g = [[b[j][h] + xb0 * w[j][0][h] + xb1 * w[j][1][h]
                      for h in (0, 1)] for j in range(3)]
                for h in (0, 1):
                    vals.append(jnp.maximum(
                        g[0][h] * _sig(g[1][h]) + g[2][h], 0.0))
            store22(dst, vals)

        def cheb(wc_v, bc_v):
            store22(tx1_v, prop(t0_v))
            p2 = prop(tx1_v)
            tx2 = []
            for d in range(_N):
                tx2.append(2.0 * p2[2 * d] - t0_v[pl.ds(d * _H, 16)])
                tx2.append(2.0 * p2[2 * d + 1] - t0_v[pl.ds(d * _H + 16, 16)])
            store22(tx2_v, tx2)

            blo = bc_v[pl.ds(0, 16)]
            bhi = bc_v[pl.ds(16, 16)]
            wbase = [[(iota + 16 * h) * _H + j * _H * _H
                      for h in (0, 1)] for j in range(3)]

            def mbody(k, acc):
                acc = list(acc)
                for j, src in enumerate((t0_v, tx1_v, tx2_v)):
                    wlo = plsc.load_gather(wc_v, [wbase[j][0] + k])
                    whi = plsc.load_gather(wc_v, [wbase[j][1] + k])
                    tcol = plsc.load_gather(src, [i32c + k])
                    for n in range(_N):
                        tb = _bc(tcol, n)
                        acc[2 * n] = acc[2 * n] + tb * wlo
                        acc[2 * n + 1] = acc[2 * n + 1] + tb * whi
                return tuple(acc)

            acc = lax.fori_loop(0, _H, mbody, (blo, bhi) * _N)
            store22(tc_v, [jnp.maximum(a, 0.0) for a in acc])

        i6w = jnp.minimum(iota, 5) * _H
        i6 = jnp.minimum(iota, 5)

        def tconv2(w_v, b_v):
            def t2body(k, acc):
                tcol = plsc.load_gather(tc_v, [i32c + k])
                w6 = plsc.load_gather(w_v, [i6w + k])
                return tuple(acc[m] + _bc(w6, m) * tcol for m in range(6))

            acc = lax.fori_loop(0, _H, t2body, (zero,) * 6)
            b6 = plsc.load_gather(b_v, [i6])
            g = [[acc[2 * j + o] + _bc(b6, 2 * j + o) for o in (0, 1)]
                 for j in range(3)]
            return tuple(jnp.maximum(
                g[0][o] * _sig(g[1][o]) + g[2][o], 0.0) for o in (0, 1))

        def bnorm(u0, u1, g_v, be_v):
            gv = plsc.load_gather(g_v, [ci])
            bv = plsc.load_gather(be_v, [ci])
            m = 0.5 * (u0 + u1)
            d0 = u0 - m
            d1 = u1 - m
            inv = _rsqrt(0.5 * (d0 * d0 + d1 * d1) + _EPS)
            return (jnp.maximum(d0 * inv * gv + bv, 0.0),
                    jnp.maximum(d1 * inv * gv + bv, 0.0))

        x0 = plsc.load_gather(obs_v, [ci * 2])
        x1 = plsc.load_gather(obs_v, [ci * 2 + 1])
        for (wa, ba, wc, bc, wb, bb, gg, be) in (
                (w1a_v, b1a_v, wc1_v, bc1_v, w1b_v, b1b_v, g1_v, be1_v),
                (w2a_v, b2a_v, wc2_v, bc2_v, w2b_v, b2b_v, g2_v, be2_v)):
            tconv1(x0, x1, wa, ba, t0_v)
            cheb(wc, bc)
            u0, u1 = tconv2(wb, bb)
            x0, x1 = bnorm(u0, u1, gg, be)

        w00 = plsc.load_gather(lw_v, [_full(0)])
        w01 = plsc.load_gather(lw_v, [_full(1)])
        lb = plsc.load_gather(lb_v, [_full(0)])
        ob_v[pl.ds(0, 16)] = x0 * w00 + x1 * w01 + lb
        pltpu.sync_copy(ob_v, out_h)


def _launch(*args):
    mesh = plsc.VectorSubcoreMesh(core_axis_name="c", subcore_axis_name="s")
    vm = lambda n, dt=_f32: pltpu.VMEM((n,), dt)
    scratch = [
        vm(22), vm(220, _i32),
        vm(192), vm(96), vm(3072), vm(32), vm(192), vm(6), vm(11), vm(11),
        vm(192), vm(96), vm(3072), vm(32), vm(192), vm(6), vm(11), vm(11),
        vm(2), vm(1),
        vm(_N * _H), vm(_N * _H), vm(_N * _H), vm(_N * _H),
        vm(144), vm(144), vm(112, _i32), vm(16), vm(16),
        pltpu.SemaphoreType.DMA,
    ]
    fn = pl.kernel(
        _body,
        out_type=jax.ShapeDtypeStruct((16,), _f32),
        mesh=mesh,
        scratch_types=scratch,
        compiler_params=pltpu.CompilerParams(needs_layout_passes=False),
    )
    return fn(*args)


def kernel(obs, edge_index,
           s1_tc1_W, s1_tc1_b, s1_cheb_W, s1_cheb_b, s1_tc2_W, s1_tc2_b,
           s1_bn_g, s1_bn_b,
           s2_tc1_W, s2_tc1_b, s2_cheb_W, s2_cheb_b, s2_tc2_W, s2_tc2_b,
           s2_bn_g, s2_bn_b, lin_W, lin_b):
    flat = [jnp.ravel(a) for a in (
        obs, edge_index,
        s1_tc1_W, s1_tc1_b, s1_cheb_W, s1_cheb_b, s1_tc2_W, s1_tc2_b,
        s1_bn_g, s1_bn_b,
        s2_tc1_W, s2_tc1_b, s2_cheb_W, s2_cheb_b, s2_tc2_W, s2_tc2_b,
        s2_bn_g, s2_bn_b, lin_W, lin_b)]
    out16 = _launch(*flat)
    return out16[1:_N]


# speedup vs baseline: 2.0549x; 1.0313x over previous
import functools

import jax
import jax.numpy as jnp
from jax import lax
from jax.experimental import pallas as pl
from jax.experimental.pallas import tpu as pltpu
from jax.experimental.pallas import tpu_sc as plsc

_N = 11
_E = 110
_H = 32
_EPS = 1e-5
_L = 16

_f32 = jnp.float32
_i32 = jnp.int32

_GDN = lax.GatherDimensionNumbers(
    offset_dims=(), collapsed_slice_dims=(0,), start_index_map=(0,))


def _bc(v, j):
    idx = jnp.full((_L, 1), j, _i32)
    return lax.gather(v, idx, dimension_numbers=_GDN, slice_sizes=(1,),
                      mode=lax.GatherScatterMode.PROMISE_IN_BOUNDS)


def _full(x):
    return jnp.full((_L,), x, _i32)


def _rsqrt(x):
    i = lax.bitcast_convert_type(x, _i32)
    i = jnp.int32(0x5F3759DF) - lax.shift_right_logical(i, 1)
    y = lax.bitcast_convert_type(i, _f32)
    for _ in range(3):
        y = y * (1.5 - 0.5 * x * y * y)
    return y


def _sig(z):
    return 1.0 / (1.0 + jnp.exp(-z))


def _body(obs_h, ei_h,
          w1a_h, b1a_h, wc1_h, bc1_h, w1b_h, b1b_h, g1_h, be1_h,
          w2a_h, b2a_h, wc2_h, bc2_h, w2b_h, b2b_h, g2_h, be2_h,
          lw_h, lb_h, out_h,
          obs_v, ei_v,
          w1a_v, b1a_v, wc1_v, bc1_v, w1b_v, b1b_v, g1_v, be1_v,
          w2a_v, b2a_v, wc2_v, bc2_v, w2b_v, b2b_v, g2_v, be2_v,
          lw_v, lb_v,
          t0_v, tx1_v, tx2_v, tc_v, cnt_v, ah_v, idx_v, dis_v, ob_v,
          sem):
    c_id = lax.axis_index("c")
    s_id = lax.axis_index("s")

    @pl.when(jnp.logical_and(c_id == 0, s_id == 0))
    def _run():
        iota = lax.iota(_i32, _L)
        ci = jnp.minimum(iota, _N - 1)
        i12 = jnp.minimum(iota, 11) * 12
        i32c = ci * _H
        zero = jnp.zeros((_L,), _f32)
        ones = jnp.ones((_L,), _f32)
        lane0 = iota == 0

        cp_ei = pltpu.async_copy(ei_h, ei_v, sem)
        pairs = ((obs_h, obs_v),
                 (w1a_h, w1a_v), (b1a_h, b1a_v), (wc1_h, wc1_v),
                 (bc1_h, bc1_v), (w1b_h, w1b_v), (b1b_h, b1b_v),
                 (g1_h, g1_v), (be1_h, be1_v),
                 (w2a_h, w2a_v), (b2a_h, b2a_v), (wc2_h, wc2_v),
                 (bc2_h, bc2_v), (w2b_h, w2b_v), (b2b_h, b2b_v),
                 (g2_h, g2_v), (be2_h, be2_v),
                 (lw_h, lw_v), (lb_h, lb_v))
        cps = [pltpu.async_copy(src, dst, sem) for src, dst in pairs]
        cp_ei.wait()

        for c in range(7):
            eidx = jnp.minimum(iota + 16 * c, _E - 1)
            r = plsc.load_gather(ei_v, [eidx])
            co = plsc.load_gather(ei_v, [eidx + _E])
            idx_v[pl.ds(16 * c, 16)] = co * 12 + r

        for v in range(9):
            cnt_v[pl.ds(16 * v, 16)] = zero

        def _cbody(e, carry):
            i = plsc.load_gather(idx_v, [_full(0) + e])
            plsc.addupdate_scatter(cnt_v, [i], ones, mask=lane0)
            return carry

        lax.fori_loop(0, _E, _cbody, 0, unroll=10)

        deg = zero
        for d in range(_N):
            deg = deg + plsc.load_gather(cnt_v, [_full(d * 12) + iota])
        dis_v[pl.ds(0, 16)] = jnp.where(deg > 0.0, _rsqrt(deg), 0.0)

        for v in range(9):
            l = iota + 16 * v
            dd = lax.div(l, _full(12))
            ss = lax.rem(l, _full(12))
            a = -(plsc.load_gather(dis_v, [dd]) * plsc.load_gather(dis_v, [ss]))
            ah_v[pl.ds(16 * v, 16)] = a * cnt_v[pl.ds(16 * v, 16)]

        for cp in cps:
            cp.wait()

        def store22(ref, vals):
            for d in range(_N):
                ref[pl.ds(d * _H, 16)] = vals[2 * d]
                ref[pl.ds(d * _H + 16, 16)] = vals[2 * d + 1]

        def prop_half(src, h):
            def pbody(s, acc):
                acol = plsc.load_gather(ah_v, [i12 + s])
                v = src[pl.ds(s * _H + 16 * h, 16)]
                out = []
                for d in range(_N):
                    if d % 2:
                        ab = _bc(acol, d)
                    else:
                        ab = plsc.load_gather(ah_v, [_full(d * 12) + s])
                    out.append(acc[d] + ab * v)
                return tuple(out)

            return lax.fori_loop(0, _N, pbody, (zero,) * _N)

        def tconv1(x0, x1, w_v, b_v, dst):
            w = [[[plsc.load_gather(w_v, [(iota + 16 * h) * 2 + (j * 64 + c)])
                   for h in (0, 1)] for c in (0, 1)] for j in range(3)]
            b = [[b_v[pl.ds(j * _H + 16 * h, 16)] for h in (0, 1)]
                 for j in range(3)]
            vals = []
            for n in range(_N):
                xb0 = _bc(x0, n)
                xb1 = _bc(x1, n)
                g = [[b[j][h] + xb0 * w[j][0][h] + xb1 * w[j][1][h]
                      for h in (0, 1)] for j in range(3)]
                for h in (0, 1):
                    vals.append(jnp.maximum(
                        g[0][h] * _sig(g[1][h]) + g[2][h], 0.0))
            store22(dst, vals)

        def cheb(wc_v, bc_v):
            for h in (0, 1):
                p1 = prop_half(t0_v, h)
                for d in range(_N):
                    tx1_v[pl.ds(d * _H + 16 * h, 16)] = p1[d]
            for h in (0, 1):
                p2 = prop_half(tx1_v, h)
                for d in range(_N):
                    tx2_v[pl.ds(d * _H + 16 * h, 16)] = (
                        2.0 * p2[d] - t0_v[pl.ds(d * _H + 16 * h, 16)])

            srcs = (t0_v, tx1_v, tx2_v)
            for h in (0, 1):
                wb = [(iota + 16 * h) * _H + j * _H * _H for j in range(3)]

                def mbody(k, acc, _wb=wb):
                    acc = list(acc)
                    for j, src in enumerate(srcs):
                        w = plsc.load_gather(wc_v, [_wb[j] + k])
                        tcol = plsc.load_gather(src, [i32c + k])
                        for n in range(_N):
                            if n % 2:
                                tb = _bc(tcol, n)
                            else:
                                tb = plsc.load_gather(src, [_full(n * _H) + k])
                            acc[n] = acc[n] + tb * w
                    return tuple(acc)

                b = bc_v[pl.ds(16 * h, 16)]
                acc = lax.fori_loop(0, _H, mbody, (b,) * _N)
                for n in range(_N):
                    tc_v[pl.ds(n * _H + 16 * h, 16)] = jnp.maximum(acc[n], 0.0)

        i6w = jnp.minimum(iota, 5) * _H
        i6 = jnp.minimum(iota, 5)

        def tconv2(w_v, b_v):
            def t2body(k, acc):
                tcol = plsc.load_gather(tc_v, [i32c + k])
                return tuple(
                    acc[m] + plsc.load_gather(w_v, [_full(m * _H) + k]) * tcol
                    for m in range(6))

            acc = lax.fori_loop(0, _H, t2body, (zero,) * 6)
            b6 = plsc.load_gather(b_v, [i6])
            g = [[acc[2 * j + o] + _bc(b6, 2 * j + o) for o in (0, 1)]
                 for j in range(3)]
            return tuple(jnp.maximum(
                g[0][o] * _sig(g[1][o]) + g[2][o], 0.0) for o in (0, 1))

        def bnorm(u0, u1, g_v, be_v):
            gv = plsc.load_gather(g_v, [ci])
            bv = plsc.load_gather(be_v, [ci])
            m = 0.5 * (u0 + u1)
            d0 = u0 - m
            d1 = u1 - m
            inv = _rsqrt(0.5 * (d0 * d0 + d1 * d1) + _EPS)
            return (jnp.maximum(d0 * inv * gv + bv, 0.0),
                    jnp.maximum(d1 * inv * gv + bv, 0.0))

        x0 = plsc.load_gather(obs_v, [ci * 2])
        x1 = plsc.load_gather(obs_v, [ci * 2 + 1])
        for (wa, ba, wc, bc, wb, bb, gg, be) in (
                (w1a_v, b1a_v, wc1_v, bc1_v, w1b_v, b1b_v, g1_v, be1_v),
                (w2a_v, b2a_v, wc2_v, bc2_v, w2b_v, b2b_v, g2_v, be2_v)):
            tconv1(x0, x1, wa, ba, t0_v)
            cheb(wc, bc)
            u0, u1 = tconv2(wb, bb)
            x0, x1 = bnorm(u0, u1, gg, be)

        w00 = plsc.load_gather(lw_v, [_full(0)])
        w01 = plsc.load_gather(lw_v, [_full(1)])
        lb = plsc.load_gather(lb_v, [_full(0)])
        ob_v[pl.ds(0, 16)] = x0 * w00 + x1 * w01 + lb
        pltpu.sync_copy(ob_v, out_h)


def _launch(*args):
    mesh = plsc.VectorSubcoreMesh(core_axis_name="c", subcore_axis_name="s",
                                  num_cores=1)
    vm = lambda n, dt=_f32: pltpu.VMEM((n,), dt)
    scratch = [
        vm(22), vm(220, _i32),
        vm(192), vm(96), vm(3072), vm(32), vm(192), vm(6), vm(11), vm(11),
        vm(192), vm(96), vm(3072), vm(32), vm(192), vm(6), vm(11), vm(11),
        vm(2), vm(1),
        vm(_N * _H), vm(_N * _H), vm(_N * _H), vm(_N * _H),
        vm(144), vm(144), vm(112, _i32), vm(16), vm(16),
        pltpu.SemaphoreType.DMA,
    ]
    fn = pl.kernel(
        _body,
        out_type=jax.ShapeDtypeStruct((16,), _f32),
        mesh=mesh,
        scratch_types=scratch,
        compiler_params=pltpu.CompilerParams(needs_layout_passes=False),
    )
    return fn(*args)


def kernel(obs, edge_index,
           s1_tc1_W, s1_tc1_b, s1_cheb_W, s1_cheb_b, s1_tc2_W, s1_tc2_b,
           s1_bn_g, s1_bn_b,
           s2_tc1_W, s2_tc1_b, s2_cheb_W, s2_cheb_b, s2_tc2_W, s2_tc2_b,
           s2_bn_g, s2_bn_b, lin_W, lin_b):
    flat = [jnp.ravel(a) for a in (
        obs, edge_index,
        s1_tc1_W, s1_tc1_b, s1_cheb_W, s1_cheb_b, s1_tc2_W, s1_tc2_b,
        s1_bn_g, s1_bn_b,
        s2_tc1_W, s2_tc1_b, s2_cheb_W, s2_cheb_b, s2_tc2_W, s2_tc2_b,
        s2_bn_g, s2_bn_b, lin_W, lin_b)]
    out16 = _launch(*flat)
    return out16[1:_N]


# speedup vs baseline: 2.0810x; 1.0127x over previous
import jax
import jax.numpy as jnp
from jax import lax
from jax.experimental import pallas as pl
from jax.experimental.pallas import tpu as pltpu
from jax.experimental.pallas import tpu_sc as plsc

_N = 11
_E = 110
_H = 32
_EPS = 1e-5
_L = 16

_f32 = jnp.float32
_i32 = jnp.int32

_SECS = (
    ("obs", 22),
    ("w1a", 192), ("b1a", 96), ("wc1", 3072), ("bc1", 32),
    ("w1b", 192), ("b1b", 6), ("g1", 11), ("be1", 11),
    ("w2a", 192), ("b2a", 96), ("wc2", 3072), ("bc2", 32),
    ("w2b", 192), ("b2b", 6), ("g2", 11), ("be2", 11),
    ("lin", 3),
)
_OFF = {}
_cur = 0
for _nm, _ln in _SECS:
    _OFF[_nm] = _cur
    _cur += -(-_ln // _L) * _L
_WSZ = _cur

_GDN = lax.GatherDimensionNumbers(
    offset_dims=(), collapsed_slice_dims=(0,), start_index_map=(0,))


def _bc(v, j):
    idx = jnp.full((_L, 1), j, _i32)
    return lax.gather(v, idx, dimension_numbers=_GDN, slice_sizes=(1,),
                      mode=lax.GatherScatterMode.PROMISE_IN_BOUNDS)


def _full(x):
    return jnp.full((_L,), x, _i32)


def _rsqrt(x):
    i = lax.bitcast_convert_type(x, _i32)
    i = jnp.int32(0x5F3759DF) - lax.shift_right_logical(i, 1)
    y = lax.bitcast_convert_type(i, _f32)
    for _ in range(3):
        y = y * (1.5 - 0.5 * x * y * y)
    return y


def _sig(z):
    return 1.0 / (1.0 + jnp.exp(-z))


def _body(wb_h, ib_h, out_h,
          wv, iv,
          t0_v, tx1_v, tx2_v, tc_v, cnt_v, ah_v, idx_v, dis_v, ob_v,
          sem):
    c_id = lax.axis_index("c")
    s_id = lax.axis_index("s")

    @pl.when(jnp.logical_and(c_id == 0, s_id == 0))
    def _run():
        iota = lax.iota(_i32, _L)
        ci = jnp.minimum(iota, _N - 1)
        i12 = jnp.minimum(iota, 11) * 12
        i32c = ci * _H
        zero = jnp.zeros((_L,), _f32)
        ones = jnp.ones((_L,), _f32)
        lane0 = iota == 0

        cp_i = pltpu.async_copy(ib_h, iv, sem)
        cp_w = pltpu.async_copy(wb_h, wv, sem)
        cp_i.wait()

        for c in range(7):
            eidx = jnp.minimum(iota + 16 * c, _E - 1)
            r = plsc.load_gather(iv, [eidx])
            co = plsc.load_gather(iv, [eidx + _E])
            idx_v[pl.ds(16 * c, 16)] = co * 12 + r

        for v in range(9):
            cnt_v[pl.ds(16 * v, 16)] = zero

        def _cbody(e, carry):
            i = plsc.load_gather(idx_v, [_full(0) + e])
            plsc.addupdate_scatter(cnt_v, [i], ones, mask=lane0)
            return carry

        lax.fori_loop(0, _E, _cbody, 0, unroll=10)

        deg = zero
        for d in range(_N):
            deg = deg + plsc.load_gather(cnt_v, [_full(d * 12) + iota])
        dis_v[pl.ds(0, 16)] = jnp.where(deg > 0.0, _rsqrt(deg), 0.0)

        for v in range(9):
            l = iota + 16 * v
            dd = lax.div(l, _full(12))
            ss = lax.rem(l, _full(12))
            a = -(plsc.load_gather(dis_v, [dd]) * plsc.load_gather(dis_v, [ss]))
            ah_v[pl.ds(16 * v, 16)] = a * cnt_v[pl.ds(16 * v, 16)]

        cp_w.wait()

        def prop_half(src, h):
            def pbody(s, acc):
                acol = plsc.load_gather(ah_v, [i12 + s])
                v = src[pl.ds(s * _H + 16 * h, 16)]
                out = []
                for d in range(_N):
                    if d % 2:
                        ab = _bc(acol, d)
                    else:
                        ab = plsc.load_gather(ah_v, [_full(d * 12) + s])
                    out.append(acc[d] + ab * v)
                return tuple(out)

            return lax.fori_loop(0, _N, pbody, (zero,) * _N)

        def tconv1(x0, x1, ow, ob, dst):
            w = [[[plsc.load_gather(wv, [(iota + 16 * h) * 2 + (ow + j * 64 + c)])
                   for h in (0, 1)] for c in (0, 1)] for j in range(3)]
            b = [[wv[pl.ds(ob + j * _H + 16 * h, 16)] for h in (0, 1)]
                 for j in range(3)]
            for n in range(_N):
                xb0 = _bc(x0, n)
                xb1 = _bc(x1, n)
                g = [[b[j][h] + xb0 * w[j][0][h] + xb1 * w[j][1][h]
                      for h in (0, 1)] for j in range(3)]
                for h in (0, 1):
                    dst[pl.ds(n * _H + 16 * h, 16)] = jnp.maximum(
                        g[0][h] * _sig(g[1][h]) + g[2][h], 0.0)

        def cheb(ow, ob):
            for h in (0, 1):
                p1 = prop_half(t0_v, h)
                for d in range(_N):
                    tx1_v[pl.ds(d * _H + 16 * h, 16)] = p1[d]
            for h in (0, 1):
                p2 = prop_half(tx1_v, h)
                for d in range(_N):
                    tx2_v[pl.ds(d * _H + 16 * h, 16)] = (
                        2.0 * p2[d] - t0_v[pl.ds(d * _H + 16 * h, 16)])

            srcs = (t0_v, tx1_v, tx2_v)
            for h in (0, 1):
                wb = [ow + (iota + 16 * h) * _H + j * _H * _H for j in range(3)]

                def mbody(k, acc, _wb=wb):
                    acc = list(acc)
                    for j, src in enumerate(srcs):
                        w = plsc.load_gather(wv, [_wb[j] + k])
                        tcol = plsc.load_gather(src, [i32c + k])
                        for n in range(_N):
                            if n % 2:
                                tb = _bc(tcol, n)
                            else:
                                tb = plsc.load_gather(src, [_full(n * _H) + k])
                            acc[n] = acc[n] + tb * w
                    return tuple(acc)

                b = wv[pl.ds(ob + 16 * h, 16)]
                acc = lax.fori_loop(0, _H, mbody, (b,) * _N)
                for n in range(_N):
                    tc_v[pl.ds(n * _H + 16 * h, 16)] = jnp.maximum(acc[n], 0.0)

        i6 = jnp.minimum(iota, 5)

        def tconv2(ow, ob):
            def t2body(k, acc):
                tcol = plsc.load_gather(tc_v, [i32c + k])
                return tuple(
                    acc[m] + plsc.load_gather(wv, [_full(ow + m * _H) + k]) * tcol
                    for m in range(6))

            acc = lax.fori_loop(0, _H, t2body, (zero,) * 6)
            b6 = plsc.load_gather(wv, [i6 + ob])
            g = [[acc[2 * j + o] + _bc(b6, 2 * j + o) for o in (0, 1)]
                 for j in range(3)]
            return tuple(jnp.maximum(
                g[0][o] * _sig(g[1][o]) + g[2][o], 0.0) for o in (0, 1))

        def bnorm(u0, u1, og, obe):
            gv = plsc.load_gather(wv, [ci + og])
            bv = plsc.load_gather(wv, [ci + obe])
            m = 0.5 * (u0 + u1)
            d0 = u0 - m
            d1 = u1 - m
            inv = _rsqrt(0.5 * (d0 * d0 + d1 * d1) + _EPS)
            return (jnp.maximum(d0 * inv * gv + bv, 0.0),
                    jnp.maximum(d1 * inv * gv + bv, 0.0))

        x0 = plsc.load_gather(wv, [ci * 2 + _OFF["obs"]])
        x1 = plsc.load_gather(wv, [ci * 2 + (_OFF["obs"] + 1)])
        for (wa, ba, wc, bc, wb2, bb, gg, be) in (
                ("w1a", "b1a", "wc1", "bc1", "w1b", "b1b", "g1", "be1"),
                ("w2a", "b2a", "wc2", "bc2", "w2b", "b2b", "g2", "be2")):
            tconv1(x0, x1, _OFF[wa], _OFF[ba], t0_v)
            cheb(_OFF[wc], _OFF[bc])
            u0, u1 = tconv2(_OFF[wb2], _OFF[bb])
            x0, x1 = bnorm(u0, u1, _OFF[gg], _OFF[be])

        lv = wv[pl.ds(_OFF["lin"], 16)]
        ob_v[pl.ds(0, 16)] = x0 * _bc(lv, 0) + x1 * _bc(lv, 1) + _bc(lv, 2)
        pltpu.sync_copy(ob_v, out_h)


def _launch(wblob, iblob):
    mesh = plsc.VectorSubcoreMesh(core_axis_name="c", subcore_axis_name="s",
                                  num_cores=1)
    vm = lambda n, dt=_f32: pltpu.VMEM((n,), dt)
    scratch = [
        vm(_WSZ), vm(2 * _E, _i32),
        vm(_N * _H), vm(_N * _H), vm(_N * _H), vm(_N * _H),
        vm(144), vm(144), vm(112, _i32), vm(16), vm(16),
        pltpu.SemaphoreType.DMA,
    ]
    fn = pl.kernel(
        _body,
        out_type=jax.ShapeDtypeStruct((16,), _f32),
        mesh=mesh,
        scratch_types=scratch,
        compiler_params=pltpu.CompilerParams(needs_layout_passes=False),
    )
    return fn(wblob, iblob)


def kernel(obs, edge_index,
           s1_tc1_W, s1_tc1_b, s1_cheb_W, s1_cheb_b, s1_tc2_W, s1_tc2_b,
           s1_bn_g, s1_bn_b,
           s2_tc1_W, s2_tc1_b, s2_cheb_W, s2_cheb_b, s2_tc2_W, s2_tc2_b,
           s2_bn_g, s2_bn_b, lin_W, lin_b):
    arrs = {
        "obs": obs,
        "w1a": s1_tc1_W, "b1a": s1_tc1_b, "wc1": s1_cheb_W, "bc1": s1_cheb_b,
        "w1b": s1_tc2_W, "b1b": s1_tc2_b, "g1": s1_bn_g, "be1": s1_bn_b,
        "w2a": s2_tc1_W, "b2a": s2_tc1_b, "wc2": s2_cheb_W, "bc2": s2_cheb_b,
        "w2b": s2_tc2_W, "b2b": s2_tc2_b, "g2": s2_bn_g, "be2": s2_bn_b,
        "lin": jnp.concatenate([jnp.ravel(lin_W), jnp.ravel(lin_b)]),
    }
    parts = []
    for nm, ln in _SECS:
        flat = jnp.ravel(arrs[nm]).astype(_f32)
        parts.append(flat)
        pad = -(-ln // _L) * _L - ln
        if pad:
            parts.append(jnp.zeros((pad,), _f32))
    wblob = jnp.concatenate(parts)
    out16 = _launch(wblob, jnp.ravel(edge_index))
    return out16[1:_N]


# speedup vs baseline: 2.4959x; 1.1994x over previous
import jax
import jax.numpy as jnp
from jax import lax
from jax.experimental import pallas as pl
from jax.experimental.pallas import tpu as pltpu
from jax.experimental.pallas import tpu_sc as plsc

_N = 11
_E = 110
_H = 32
_EPS = 1e-5
_L = 16
_SA = 33
_SG = 13

_f32 = jnp.float32
_i32 = jnp.int32

_SECS = (
    ("obs", 22),
    ("w1a", 192), ("b1a", 96), ("wc1", 3072), ("bc1", 32),
    ("w1b", 512), ("b1b", 6), ("g1", 11), ("be1", 11),
    ("w2a", 192), ("b2a", 96), ("wc2", 3072), ("bc2", 32),
    ("w2b", 512), ("b2b", 6), ("g2", 11), ("be2", 11),
    ("lin", 3),
)
_OFF = {}
_cur = 0
for _nm, _ln in _SECS:
    _OFF[_nm] = _cur
    _cur += -(-_ln // _L) * _L
_WSZ = _cur

_GDN = lax.GatherDimensionNumbers(
    offset_dims=(), collapsed_slice_dims=(0,), start_index_map=(0,))


def _dg(v, idx):
    return lax.gather(v, idx[:, None], dimension_numbers=_GDN,
                      slice_sizes=(1,),
                      mode=lax.GatherScatterMode.PROMISE_IN_BOUNDS)


def _bc(v, j):
    return _dg(v, jnp.full((_L,), j, _i32))


def _full(x):
    return jnp.full((_L,), x, _i32)


def _rsqrt(x):
    i = lax.bitcast_convert_type(x, _i32)
    i = jnp.int32(0x5F3759DF) - lax.shift_right_logical(i, 1)
    y = lax.bitcast_convert_type(i, _f32)
    for _ in range(3):
        y = y * (1.5 - 0.5 * x * y * y)
    return y


def _sig(z):
    return 1.0 / (1.0 + jnp.exp(-z))


def _body(wb_h, ib_h, out_h,
          wv, iv,
          t0_v, tx1_v, tx2_v, tc_v, cnt_v, ah_v, ob_v,
          sem):
    c_id = lax.axis_index("c")
    s_id = lax.axis_index("s")

    @pl.when(jnp.logical_and(c_id == 0, s_id == 0))
    def _run():
        iota = lax.iota(_i32, _L)
        ci = jnp.minimum(iota, _N - 1)
        i13 = jnp.minimum(iota, 11) * _SG
        i33 = ci * _SA
        zero = jnp.zeros((_L,), _f32)
        ones = jnp.ones((_L,), _f32)
        lane0 = iota == 0

        cp_i = pltpu.async_copy(ib_h, iv, sem)
        cp_w = pltpu.async_copy(wb_h, wv, sem)
        cp_i.wait()

        for v in range(10):
            cnt_v[pl.ds(16 * v, 16)] = zero

        for c in range(7):
            r = iv[pl.ds(16 * c, 16)]
            co = iv[pl.ds(_E + 16 * c, 16)]
            ivec = co * _SG + r
            for l in range(16 if c < 6 else _E - 96):
                plsc.addupdate_scatter(cnt_v, [_bc(ivec, l)], ones,
                                       mask=lane0)

        deg = zero
        for d in range(_N):
            deg = deg + plsc.load_gather(cnt_v, [_full(d * _SG) + iota])
        dis = jnp.where(deg > 0.0, _rsqrt(deg), 0.0)

        for v in range(10):
            l = iota + 16 * v
            dd = lax.div(l, _full(_SG))
            ss = lax.rem(l, _full(_SG))
            a = -(_dg(dis, jnp.minimum(dd, 12)) * _dg(dis, ss))
            ah_v[pl.ds(16 * v, 16)] = a * cnt_v[pl.ds(16 * v, 16)]

        cp_w.wait()

        def prop(src):
            def pbody(s, acc):
                acol = plsc.load_gather(ah_v, [i13 + s])
                vlo = src[pl.ds(s * _SA, 16)]
                vhi = src[pl.ds(s * _SA + 16, 16)]
                out = []
                for d in range(_N):
                    ab = _bc(acol, d)
                    out.append(acc[2 * d] + ab * vlo)
                    out.append(acc[2 * d + 1] + ab * vhi)
                return tuple(out)

            return lax.fori_loop(0, _N, pbody, (zero,) * 22)

        def tconv1(x0, x1, ow, ob, dst):
            w = [[[wv[pl.ds(ow + j * 64 + c * 32 + 16 * h, 16)]
                   for h in (0, 1)] for c in (0, 1)] for j in range(3)]
            b = [[wv[pl.ds(ob + j * _H + 16 * h, 16)] for h in (0, 1)]
                 for j in range(3)]
            for n in range(_N):
                xb0 = _bc(x0, n)
                xb1 = _bc(x1, n)
                g = [[b[j][h] + xb0 * w[j][0][h] + xb1 * w[j][1][h]
                      for h in (0, 1)] for j in range(3)]
                for h in (0, 1):
                    dst[pl.ds(n * _SA + 16 * h, 16)] = jnp.maximum(
                        g[0][h] * _sig(g[1][h]) + g[2][h], 0.0)

        def cheb(ow, ob):
            p1 = prop(t0_v)
            for d in range(_N):
                tx1_v[pl.ds(d * _SA, 16)] = p1[2 * d]
                tx1_v[pl.ds(d * _SA + 16, 16)] = p1[2 * d + 1]
            p2 = prop(tx1_v)
            for d in range(_N):
                tx2_v[pl.ds(d * _SA, 16)] = (
                    2.0 * p2[2 * d] - t0_v[pl.ds(d * _SA, 16)])
                tx2_v[pl.ds(d * _SA + 16, 16)] = (
                    2.0 * p2[2 * d + 1] - t0_v[pl.ds(d * _SA + 16, 16)])

            acc = tuple(wv[pl.ds(ob + 16 * (i % 2), 16)] for i in range(22))
            for j, src in enumerate((t0_v, tx1_v, tx2_v)):

                def mbody(k, acc, _j=j, _src=src):
                    acc = list(acc)
                    wlo = wv[pl.ds(ow + _j * 1024 + k * _H, 16)]
                    whi = wv[pl.ds(ow + _j * 1024 + k * _H + 16, 16)]
                    tcol = plsc.load_gather(_src, [i33 + k])
                    for n in range(_N):
                        tb = _bc(tcol, n)
                        acc[2 * n] = acc[2 * n] + tb * wlo
                        acc[2 * n + 1] = acc[2 * n + 1] + tb * whi
                    return tuple(acc)

                acc = lax.fori_loop(0, _H, mbody, acc)
            for n in range(_N):
                tc_v[pl.ds(n * _SA, 16)] = jnp.maximum(acc[2 * n], 0.0)
                tc_v[pl.ds(n * _SA + 16, 16)] = jnp.maximum(acc[2 * n + 1], 0.0)

        def tconv2(ow, ob):
            def t2body(k, acc):
                tcol = plsc.load_gather(tc_v, [i33 + k])
                w6 = wv[pl.ds(ow + k * _L, 16)]
                return tuple(acc[m] + _bc(w6, m) * tcol for m in range(6))

            acc = lax.fori_loop(0, _H, t2body, (zero,) * 6)
            b6 = wv[pl.ds(ob, 16)]
            g = [[acc[2 * j + o] + _bc(b6, 2 * j + o) for o in (0, 1)]
                 for j in range(3)]
            return tuple(jnp.maximum(
                g[0][o] * _sig(g[1][o]) + g[2][o], 0.0) for o in (0, 1))

        def bnorm(u0, u1, og, obe):
            gv = wv[pl.ds(og, 16)]
            bv = wv[pl.ds(obe, 16)]
            m = 0.5 * (u0 + u1)
            d0 = u0 - m
            d1 = u1 - m
            inv = _rsqrt(0.5 * (d0 * d0 + d1 * d1) + _EPS)
            return (jnp.maximum(d0 * inv * gv + bv, 0.0),
                    jnp.maximum(d1 * inv * gv + bv, 0.0))

        x0 = wv[pl.ds(_OFF["obs"], 16)]
        x1 = wv[pl.ds(_OFF["obs"] + _N, 16)]
        for (wa, ba, wc, bc, wb2, bb, gg, be) in (
                ("w1a", "b1a", "wc1", "bc1", "w1b", "b1b", "g1", "be1"),
                ("w2a", "b2a", "wc2", "bc2", "w2b", "b2b", "g2", "be2")):
            tconv1(x0, x1, _OFF[wa], _OFF[ba], t0_v)
            cheb(_OFF[wc], _OFF[bc])
            u0, u1 = tconv2(_OFF[wb2], _OFF[bb])
            x0, x1 = bnorm(u0, u1, _OFF[gg], _OFF[be])

        lv = wv[pl.ds(_OFF["lin"], 16)]
        ob_v[pl.ds(0, 16)] = x0 * _bc(lv, 0) + x1 * _bc(lv, 1) + _bc(lv, 2)
        pltpu.sync_copy(ob_v, out_h)


def _launch(wblob, iblob):
    mesh = plsc.VectorSubcoreMesh(core_axis_name="c", subcore_axis_name="s",
                                  num_cores=1)
    vm = lambda n, dt=_f32: pltpu.VMEM((n,), dt)
    scratch = [
        vm(_WSZ), vm(224, _i32),
        vm(368), vm(368), vm(368), vm(368),
        vm(160), vm(160), vm(16),
        pltpu.SemaphoreType.DMA,
    ]
    fn = pl.kernel(
        _body,
        out_type=jax.ShapeDtypeStruct((16,), _f32),
        mesh=mesh,
        scratch_types=scratch,
        compiler_params=pltpu.CompilerParams(needs_layout_passes=False),
    )
    return fn(wblob, iblob)


def kernel(obs, edge_index,
           s1_tc1_W, s1_tc1_b, s1_cheb_W, s1_cheb_b, s1_tc2_W, s1_tc2_b,
           s1_bn_g, s1_bn_b,
           s2_tc1_W, s2_tc1_b, s2_cheb_W, s2_cheb_b, s2_tc2_W, s2_tc2_b,
           s2_bn_g, s2_bn_b, lin_W, lin_b):
    def t2pack(w):
        return jnp.pad(w.reshape(6, _H).T, ((0, 0), (0, 10)))

    arrs = {
        "obs": obs.T,
        "w1a": s1_tc1_W.transpose(0, 2, 1), "b1a": s1_tc1_b,
        "wc1": s1_cheb_W.transpose(0, 2, 1), "bc1": s1_cheb_b,
        "w1b": t2pack(s1_tc2_W), "b1b": s1_tc2_b,
        "g1": s1_bn_g, "be1": s1_bn_b,
        "w2a": s2_tc1_W.transpose(0, 2, 1), "b2a": s2_tc1_b,
        "wc2": s2_cheb_W.transpose(0, 2, 1), "bc2": s2_cheb_b,
        "w2b": t2pack(s2_tc2_W), "b2b": s2_tc2_b,
        "g2": s2_bn_g, "be2": s2_bn_b,
        "lin": jnp.concatenate([jnp.ravel(lin_W), jnp.ravel(lin_b)]),
    }
    parts = []
    for nm, ln in _SECS:
        flat = jnp.ravel(arrs[nm]).astype(_f32)
        parts.append(flat)
        pad = -(-ln // _L) * _L - ln
        if pad:
            parts.append(jnp.zeros((pad,), _f32))
    wblob = jnp.concatenate(parts)
    iblob = jnp.concatenate([jnp.ravel(edge_index),
                             jnp.zeros((4,), _i32)])
    out16 = _launch(wblob, iblob)
    return out16[1:_N]
